# initial kernel scaffold (unmeasured)
import jax
import jax.numpy as jnp
from jax import lax
from jax.experimental import pallas as pl
from jax.experimental.pallas import tpu as pltpu

N_DEV = 4
M = 4096
N = 8192
CHUNK = M // N_DEV
PIECE = 512
PIECES = CHUNK // PIECE


def _gelu(y):
    c = 0.7978845608028654
    return 0.5 * y * (1.0 + jnp.tanh(c * (y + 0.044715 * y * y * y)))


def kernel(x, w_mat):
    p = jnp.dot(x, w_mat, preferred_element_type=jnp.float32)

    def body(p_ref, out_ref, send_buf, recv_buf, va, vb,
             rs_send, rs_recv, ag_send, ag_recv, sem_a, sem_b, sem_o):
        me = lax.axis_index("i")
        right = lax.rem(me + 1, N_DEV)
        left = lax.rem(me + N_DEV - 1, N_DEV)

        barrier = pltpu.get_barrier_semaphore()
        for nbr in (left, right):
            pl.semaphore_signal(barrier, inc=1, device_id=(nbr,),
                                device_id_type=pl.DeviceIdType.MESH)
        pl.semaphore_wait(barrier, 2)

        def accum(src_chunk, p_row0, dest, dest_row0, apply_gelu):
            for j in range(PIECES):
                ca = pltpu.make_async_copy(
                    src_chunk.at[pl.ds(j * PIECE, PIECE), :], va, sem_a)
                cb = pltpu.make_async_copy(
                    p_ref.at[pl.ds(p_row0 + j * PIECE, PIECE), :], vb, sem_b)
                ca.start()
                cb.start()
                ca.wait()
                cb.wait()
                s = va[...] + vb[...]
                va[...] = _gelu(s) if apply_gelu else s
                co = pltpu.make_async_copy(
                    va, dest.at[pl.ds(dest_row0 + j * PIECE, PIECE), :], sem_o)
                co.start()
                co.wait()

        for s in range(N_DEV - 1):
            send_chunk = lax.rem(me - s + N_DEV, N_DEV)
            recv_chunk = lax.rem(me - s - 1 + N_DEV, N_DEV)
            if s == 0:
                src = p_ref.at[pl.ds(send_chunk * CHUNK, CHUNK), :]
            else:
                src = send_buf.at[s - 1]
            rdma = pltpu.make_async_remote_copy(
                src_ref=src,
                dst_ref=recv_buf.at[s],
                send_sem=rs_send.at[s],
                recv_sem=rs_recv.at[s],
                device_id=(right,),
                device_id_type=pl.DeviceIdType.MESH,
            )
            rdma.start()
            rdma.wait_recv()
            if s < N_DEV - 2:
                accum(recv_buf.at[s], recv_chunk * CHUNK, send_buf.at[s], 0,
                      apply_gelu=False)
            else:
                accum(recv_buf.at[s], recv_chunk * CHUNK, out_ref,
                      recv_chunk * CHUNK, apply_gelu=True)
            rdma.wait_send()

        for t in range(N_DEV - 1):
            g = lax.rem(me + 1 - t + N_DEV, N_DEV)
            rows = g * CHUNK
            rdma = pltpu.make_async_remote_copy(
                src_ref=out_ref.at[pl.ds(rows, CHUNK), :],
                dst_ref=out_ref.at[pl.ds(rows, CHUNK), :],
                send_sem=ag_send.at[t],
                recv_sem=ag_recv.at[t],
                device_id=(right,),
                device_id_type=pl.DeviceIdType.MESH,
            )
            rdma.start()
            rdma.wait_recv()
            rdma.wait_send()

        def _exit(second_barrier):
            for nbr in (left, right):
                pl.semaphore_signal(second_barrier, inc=1, device_id=(nbr,),
                                    device_id_type=pl.DeviceIdType.MESH)
            pl.semaphore_wait(second_barrier, 2)
        pl.run_scoped(_exit, second_barrier=pltpu.SemaphoreType.REGULAR)

    return pl.pallas_call(
        body,
        out_shape=jax.ShapeDtypeStruct((M, N), jnp.float32),
        in_specs=[pl.BlockSpec(memory_space=pl.ANY)],
        out_specs=pl.BlockSpec(memory_space=pl.ANY),
        scratch_shapes=[
            pltpu.MemorySpace.HBM((N_DEV - 2, CHUNK, N), jnp.float32),
            pltpu.MemorySpace.HBM((N_DEV - 1, CHUNK, N), jnp.float32),
            pltpu.MemorySpace.VMEM((PIECE, N), jnp.float32),
            pltpu.MemorySpace.VMEM((PIECE, N), jnp.float32),
            pltpu.SemaphoreType.DMA((N_DEV - 1,)),
            pltpu.SemaphoreType.DMA((N_DEV - 1,)),
            pltpu.SemaphoreType.DMA((N_DEV - 1,)),
            pltpu.SemaphoreType.DMA((N_DEV - 1,)),
            pltpu.SemaphoreType.DMA,
            pltpu.SemaphoreType.DMA,
            pltpu.SemaphoreType.DMA,
        ],
        compiler_params=pltpu.CompilerParams(collective_id=0),
    )(p)


# baseline (device time: 2457682 ns/iter reference)
import jax
import jax.numpy as jnp
from jax import lax
from jax.experimental import pallas as pl
from jax.experimental.pallas import tpu as pltpu

N_DEV = 4
M = 4096
N = 8192
CHUNK = M // N_DEV
PIECE = 512
PIECES = CHUNK // PIECE


def _gelu(y):
    c = 0.7978845608028654
    return 0.5 * y * (1.0 + jnp.tanh(c * (y + 0.044715 * y * y * y)))


def kernel(x, w_mat):
    p = jnp.dot(x, w_mat, preferred_element_type=jnp.float32)

    def body(p_ref, out_ref, send_buf, recv_buf, va, vb,
             rs_send, rs_recv, ag_send, ag_recv, sem_a, sem_b, sem_o):
        me = lax.axis_index("i")
        right = lax.rem(me + 1, N_DEV)
        left = lax.rem(me + N_DEV - 1, N_DEV)

        barrier = pltpu.get_barrier_semaphore()
        for nbr in (left, right):
            pl.semaphore_signal(barrier, inc=1, device_id=(nbr,),
                                device_id_type=pl.DeviceIdType.MESH)
        pl.semaphore_wait(barrier, 2)

        def accum(src_chunk, p_row0, dest, dest_row0, apply_gelu):
            for j in range(PIECES):
                ca = pltpu.make_async_copy(
                    src_chunk.at[pl.ds(j * PIECE, PIECE), :], va, sem_a)
                cb = pltpu.make_async_copy(
                    p_ref.at[pl.ds(p_row0 + j * PIECE, PIECE), :], vb, sem_b)
                ca.start()
                cb.start()
                ca.wait()
                cb.wait()
                s = va[...] + vb[...]
                va[...] = _gelu(s) if apply_gelu else s
                co = pltpu.make_async_copy(
                    va, dest.at[pl.ds(dest_row0 + j * PIECE, PIECE), :], sem_o)
                co.start()
                co.wait()

        for s in range(N_DEV - 1):
            send_chunk = lax.rem(me - s + N_DEV, N_DEV)
            recv_chunk = lax.rem(me - s - 1 + N_DEV, N_DEV)
            if s == 0:
                src = p_ref.at[pl.ds(send_chunk * CHUNK, CHUNK), :]
            else:
                src = send_buf.at[s - 1]
            rdma = pltpu.make_async_remote_copy(
                src_ref=src,
                dst_ref=recv_buf.at[s],
                send_sem=rs_send.at[s],
                recv_sem=rs_recv.at[s],
                device_id=(right,),
                device_id_type=pl.DeviceIdType.MESH,
            )
            rdma.start()
            rdma.wait_recv()
            if s < N_DEV - 2:
                accum(recv_buf.at[s], recv_chunk * CHUNK, send_buf.at[s], 0,
                      apply_gelu=False)
            else:
                accum(recv_buf.at[s], recv_chunk * CHUNK, out_ref,
                      recv_chunk * CHUNK, apply_gelu=True)
            rdma.wait_send()

        for t in range(N_DEV - 1):
            g = lax.rem(me + 1 - t + N_DEV, N_DEV)
            rows = g * CHUNK
            rdma = pltpu.make_async_remote_copy(
                src_ref=out_ref.at[pl.ds(rows, CHUNK), :],
                dst_ref=out_ref.at[pl.ds(rows, CHUNK), :],
                send_sem=ag_send.at[t],
                recv_sem=ag_recv.at[t],
                device_id=(right,),
                device_id_type=pl.DeviceIdType.MESH,
            )
            rdma.start()
            rdma.wait_recv()
            rdma.wait_send()

        def _exit(second_barrier):
            for nbr in (left, right):
                pl.semaphore_signal(second_barrier, inc=1, device_id=(nbr,),
                                    device_id_type=pl.DeviceIdType.MESH)
            pl.semaphore_wait(second_barrier, 2)
        pl.run_scoped(_exit, second_barrier=pltpu.SemaphoreType.REGULAR)

    out = pl.pallas_call(
        body,
        out_shape=[
            jax.ShapeDtypeStruct((M, N), jnp.float32),
            jax.ShapeDtypeStruct((N_DEV - 2, CHUNK, N), jnp.float32),
            jax.ShapeDtypeStruct((N_DEV - 1, CHUNK, N), jnp.float32),
        ],
        in_specs=[pl.BlockSpec(memory_space=pl.ANY)],
        out_specs=[pl.BlockSpec(memory_space=pl.ANY)] * 3,
        scratch_shapes=[
            pltpu.MemorySpace.VMEM((PIECE, N), jnp.float32),
            pltpu.MemorySpace.VMEM((PIECE, N), jnp.float32),
            pltpu.SemaphoreType.DMA((N_DEV - 1,)),
            pltpu.SemaphoreType.DMA((N_DEV - 1,)),
            pltpu.SemaphoreType.DMA((N_DEV - 1,)),
            pltpu.SemaphoreType.DMA((N_DEV - 1,)),
            pltpu.SemaphoreType.DMA,
            pltpu.SemaphoreType.DMA,
            pltpu.SemaphoreType.DMA,
        ],
        compiler_params=pltpu.CompilerParams(collective_id=0),
    )(p)
    return out[0]


# device time: 1393127 ns/iter; 1.7641x vs baseline; 1.7641x over previous
import jax
import jax.numpy as jnp
from jax import lax
from jax.experimental import pallas as pl
from jax.experimental.pallas import tpu as pltpu

N_DEV = 4
M = 4096
N = 8192
HALF = N // 2
CHUNK = M // N_DEV
PIECE = 512
PIECES = CHUNK // PIECE


def _gelu(y):
    c = 0.7978845608028654
    return 0.5 * y * (1.0 + jnp.tanh(c * (y + 0.044715 * y * y * y)))


def kernel(x, w_mat):
    p = jnp.dot(x, w_mat, preferred_element_type=jnp.float32)

    def body(p_ref, out_ref, sb0, sb1, rb0, rb1, va, vb,
             rs_send, rs_recv, ag_send, ag_recv, sem_a, sem_b, sem_o):
        me = lax.axis_index("i")
        right = lax.rem(me + 1, N_DEV)
        left = lax.rem(me + N_DEV - 1, N_DEV)

        barrier = pltpu.get_barrier_semaphore()
        for nbr in (left, right):
            pl.semaphore_signal(barrier, inc=1, device_id=(nbr,),
                                device_id_type=pl.DeviceIdType.MESH)
        pl.semaphore_wait(barrier, 2)

        def accum(src_chunk, p_row0, p_col0, dest, dest_row0, dest_col0,
                  apply_gelu):
            for j in range(PIECES):
                ca = pltpu.make_async_copy(
                    src_chunk.at[pl.ds(j * PIECE, PIECE), :], va, sem_a)
                cb = pltpu.make_async_copy(
                    p_ref.at[pl.ds(p_row0 + j * PIECE, PIECE),
                             pl.ds(p_col0, HALF)], vb, sem_b)
                ca.start()
                cb.start()
                ca.wait()
                cb.wait()
                s = va[...] + vb[...]
                va[...] = _gelu(s) if apply_gelu else s
                co = pltpu.make_async_copy(
                    va, dest.at[pl.ds(dest_row0 + j * PIECE, PIECE),
                                pl.ds(dest_col0, HALF)], sem_o)
                co.start()
                co.wait()

        for s in range(N_DEV - 1):
            c0 = lax.rem(me - s + N_DEV, N_DEV)
            r0 = lax.rem(me - s - 1 + N_DEV, N_DEV)
            c1 = lax.rem(me + s, N_DEV)
            r1 = lax.rem(me + s + 1, N_DEV)
            if s == 0:
                src0 = p_ref.at[pl.ds(c0 * CHUNK, CHUNK), pl.ds(0, HALF)]
                src1 = p_ref.at[pl.ds(c1 * CHUNK, CHUNK), pl.ds(HALF, HALF)]
            else:
                src0 = sb0.at[s - 1]
                src1 = sb1.at[s - 1]
            rdma0 = pltpu.make_async_remote_copy(
                src_ref=src0, dst_ref=rb0.at[s],
                send_sem=rs_send.at[0, s], recv_sem=rs_recv.at[0, s],
                device_id=(right,), device_id_type=pl.DeviceIdType.MESH,
            )
            rdma1 = pltpu.make_async_remote_copy(
                src_ref=src1, dst_ref=rb1.at[s],
                send_sem=rs_send.at[1, s], recv_sem=rs_recv.at[1, s],
                device_id=(left,), device_id_type=pl.DeviceIdType.MESH,
            )
            rdma0.start()
            rdma1.start()
            last = s == N_DEV - 2
            rdma0.wait_recv()
            if not last:
                accum(rb0.at[s], r0 * CHUNK, 0, sb0.at[s], 0, 0, False)
            else:
                accum(rb0.at[s], r0 * CHUNK, 0, out_ref, r0 * CHUNK, 0, True)
            rdma1.wait_recv()
            if not last:
                accum(rb1.at[s], r1 * CHUNK, HALF, sb1.at[s], 0, 0, False)
            else:
                accum(rb1.at[s], r1 * CHUNK, HALF, out_ref, r1 * CHUNK, HALF,
                      True)
            rdma0.wait_send()
            rdma1.wait_send()

        for t in range(N_DEV - 1):
            g0 = lax.rem(me + 1 - t + N_DEV, N_DEV)
            g1 = lax.rem(me - 1 + t + N_DEV, N_DEV)
            rdma0 = pltpu.make_async_remote_copy(
                src_ref=out_ref.at[pl.ds(g0 * CHUNK, CHUNK), pl.ds(0, HALF)],
                dst_ref=out_ref.at[pl.ds(g0 * CHUNK, CHUNK), pl.ds(0, HALF)],
                send_sem=ag_send.at[0, t], recv_sem=ag_recv.at[0, t],
                device_id=(right,), device_id_type=pl.DeviceIdType.MESH,
            )
            rdma1 = pltpu.make_async_remote_copy(
                src_ref=out_ref.at[pl.ds(g1 * CHUNK, CHUNK), pl.ds(HALF, HALF)],
                dst_ref=out_ref.at[pl.ds(g1 * CHUNK, CHUNK), pl.ds(HALF, HALF)],
                send_sem=ag_send.at[1, t], recv_sem=ag_recv.at[1, t],
                device_id=(left,), device_id_type=pl.DeviceIdType.MESH,
            )
            rdma0.start()
            rdma1.start()
            rdma0.wait_recv()
            rdma1.wait_recv()
            rdma0.wait_send()
            rdma1.wait_send()

        def _exit(second_barrier):
            for nbr in (left, right):
                pl.semaphore_signal(second_barrier, inc=1, device_id=(nbr,),
                                    device_id_type=pl.DeviceIdType.MESH)
            pl.semaphore_wait(second_barrier, 2)
        pl.run_scoped(_exit, second_barrier=pltpu.SemaphoreType.REGULAR)

    out = pl.pallas_call(
        body,
        out_shape=[
            jax.ShapeDtypeStruct((M, N), jnp.float32),
            jax.ShapeDtypeStruct((N_DEV - 2, CHUNK, HALF), jnp.float32),
            jax.ShapeDtypeStruct((N_DEV - 2, CHUNK, HALF), jnp.float32),
            jax.ShapeDtypeStruct((N_DEV - 1, CHUNK, HALF), jnp.float32),
            jax.ShapeDtypeStruct((N_DEV - 1, CHUNK, HALF), jnp.float32),
        ],
        in_specs=[pl.BlockSpec(memory_space=pl.ANY)],
        out_specs=[pl.BlockSpec(memory_space=pl.ANY)] * 5,
        scratch_shapes=[
            pltpu.MemorySpace.VMEM((PIECE, HALF), jnp.float32),
            pltpu.MemorySpace.VMEM((PIECE, HALF), jnp.float32),
            pltpu.SemaphoreType.DMA((2, N_DEV - 1)),
            pltpu.SemaphoreType.DMA((2, N_DEV - 1)),
            pltpu.SemaphoreType.DMA((2, N_DEV - 1)),
            pltpu.SemaphoreType.DMA((2, N_DEV - 1)),
            pltpu.SemaphoreType.DMA,
            pltpu.SemaphoreType.DMA,
            pltpu.SemaphoreType.DMA,
        ],
        compiler_params=pltpu.CompilerParams(collective_id=0),
    )(p)
    return out[0]


# device time: 1277111 ns/iter; 1.9244x vs baseline; 1.0908x over previous
import jax
import jax.numpy as jnp
from jax import lax
from jax.experimental import pallas as pl
from jax.experimental.pallas import tpu as pltpu

N_DEV = 4
M = 4096
N = 8192
HALF = N // 2
CHUNK = M // N_DEV
PIECE = 256
PIECES = CHUNK // PIECE
N_STEP = N_DEV - 1


def _gelu(y):
    c = 0.7978845608028654
    return 0.5 * y * (1.0 + jnp.tanh(c * (y + 0.044715 * y * y * y)))


def kernel(x, w_mat):
    p = jnp.dot(x, w_mat, preferred_element_type=jnp.float32)

    def body(p_ref, out_ref, sb0, sb1, rb0, rb1, va, vb,
             rs_send, rs_recv, ag_send, ag_recv, sem_a, sem_b, sem_o):
        me = lax.axis_index("i")
        right = lax.rem(me + 1, N_DEV)
        left = lax.rem(me + N_DEV - 1, N_DEV)
        ring_nbr = (right, left)
        ring_col0 = (0, HALF)
        ring_rb = (rb0, rb1)
        ring_sb = (sb0, sb1)

        barrier = pltpu.get_barrier_semaphore()
        for nbr in (left, right):
            pl.semaphore_signal(barrier, inc=1, device_id=(nbr,),
                                device_id_type=pl.DeviceIdType.MESH)
        pl.semaphore_wait(barrier, 2)

        def accum_piece(src_piece, p_row0, p_col0, dest_piece, apply_gelu):
            ca = pltpu.make_async_copy(src_piece, va, sem_a)
            cb = pltpu.make_async_copy(
                p_ref.at[pl.ds(p_row0, PIECE), pl.ds(p_col0, HALF)], vb, sem_b)
            ca.start()
            cb.start()
            ca.wait()
            cb.wait()
            s = va[...] + vb[...]
            va[...] = _gelu(s) if apply_gelu else s
            co = pltpu.make_async_copy(va, dest_piece, sem_o)
            co.start()
            co.wait()

        descs = {}

        def start_send(r, s, j, src_piece):
            d = pltpu.make_async_remote_copy(
                src_ref=src_piece,
                dst_ref=ring_rb[r].at[s, pl.ds(j * PIECE, PIECE), :],
                send_sem=rs_send.at[r, s, j],
                recv_sem=rs_recv.at[r, s, j],
                device_id=(ring_nbr[r],),
                device_id_type=pl.DeviceIdType.MESH,
            )
            d.start()
            descs[(r, s, j)] = d

        for j in range(PIECES):
            for r in (0, 1):
                start_send(r, 0, j, p_ref.at[
                    pl.ds(me * CHUNK + j * PIECE, PIECE),
                    pl.ds(ring_col0[r], HALF)])

        for s in range(N_STEP):
            last = s == N_STEP - 1
            rc = (lax.rem(me - s - 1 + N_DEV, N_DEV),
                  lax.rem(me + s + 1, N_DEV))
            for j in range(PIECES):
                for r in (0, 1):
                    descs[(r, s, j)].wait_recv()
                    src = ring_rb[r].at[s, pl.ds(j * PIECE, PIECE), :]
                    p_row0 = rc[r] * CHUNK + j * PIECE
                    if not last:
                        dest = ring_sb[r].at[s, pl.ds(j * PIECE, PIECE), :]
                        accum_piece(src, p_row0, ring_col0[r], dest, False)
                        start_send(r, s + 1, j, dest)
                    else:
                        dest = out_ref.at[pl.ds(p_row0, PIECE),
                                          pl.ds(ring_col0[r], HALF)]
                        accum_piece(src, p_row0, ring_col0[r], dest, True)
        for d in descs.values():
            d.wait_send()

        for t in range(N_STEP):
            g0 = lax.rem(me + 1 - t + N_DEV, N_DEV)
            g1 = lax.rem(me - 1 + t + N_DEV, N_DEV)
            rdma0 = pltpu.make_async_remote_copy(
                src_ref=out_ref.at[pl.ds(g0 * CHUNK, CHUNK), pl.ds(0, HALF)],
                dst_ref=out_ref.at[pl.ds(g0 * CHUNK, CHUNK), pl.ds(0, HALF)],
                send_sem=ag_send.at[0, t], recv_sem=ag_recv.at[0, t],
                device_id=(right,), device_id_type=pl.DeviceIdType.MESH,
            )
            rdma1 = pltpu.make_async_remote_copy(
                src_ref=out_ref.at[pl.ds(g1 * CHUNK, CHUNK), pl.ds(HALF, HALF)],
                dst_ref=out_ref.at[pl.ds(g1 * CHUNK, CHUNK), pl.ds(HALF, HALF)],
                send_sem=ag_send.at[1, t], recv_sem=ag_recv.at[1, t],
                device_id=(left,), device_id_type=pl.DeviceIdType.MESH,
            )
            rdma0.start()
            rdma1.start()
            rdma0.wait_recv()
            rdma1.wait_recv()
            rdma0.wait_send()
            rdma1.wait_send()

        def _exit(second_barrier):
            for nbr in (left, right):
                pl.semaphore_signal(second_barrier, inc=1, device_id=(nbr,),
                                    device_id_type=pl.DeviceIdType.MESH)
            pl.semaphore_wait(second_barrier, 2)
        pl.run_scoped(_exit, second_barrier=pltpu.SemaphoreType.REGULAR)

    out = pl.pallas_call(
        body,
        out_shape=[
            jax.ShapeDtypeStruct((M, N), jnp.float32),
            jax.ShapeDtypeStruct((N_STEP - 1, CHUNK, HALF), jnp.float32),
            jax.ShapeDtypeStruct((N_STEP - 1, CHUNK, HALF), jnp.float32),
            jax.ShapeDtypeStruct((N_STEP, CHUNK, HALF), jnp.float32),
            jax.ShapeDtypeStruct((N_STEP, CHUNK, HALF), jnp.float32),
        ],
        in_specs=[pl.BlockSpec(memory_space=pl.ANY)],
        out_specs=[pl.BlockSpec(memory_space=pl.ANY)] * 5,
        scratch_shapes=[
            pltpu.MemorySpace.VMEM((PIECE, HALF), jnp.float32),
            pltpu.MemorySpace.VMEM((PIECE, HALF), jnp.float32),
            pltpu.SemaphoreType.DMA((2, N_STEP, PIECES)),
            pltpu.SemaphoreType.DMA((2, N_STEP, PIECES)),
            pltpu.SemaphoreType.DMA((2, N_STEP)),
            pltpu.SemaphoreType.DMA((2, N_STEP)),
            pltpu.SemaphoreType.DMA,
            pltpu.SemaphoreType.DMA,
            pltpu.SemaphoreType.DMA,
        ],
        compiler_params=pltpu.CompilerParams(collective_id=0),
    )(p)
    return out[0]


# device time: 1257082 ns/iter; 1.9551x vs baseline; 1.0159x over previous
import jax
import jax.numpy as jnp
from jax import lax
from jax.experimental import pallas as pl
from jax.experimental.pallas import tpu as pltpu

N_DEV = 4
M = 4096
N = 8192
HALF = N // 2
CHUNK = M // N_DEV
PIECE = 256
PIECES = CHUNK // PIECE
N_STEP = N_DEV - 1


def _gelu(y):
    c = 0.7978845608028654
    return 0.5 * y * (1.0 + jnp.tanh(c * (y + 0.044715 * y * y * y)))


def kernel(x, w_mat):
    p = jnp.dot(x, w_mat, preferred_element_type=jnp.float32)

    def body(p_ref, out_ref, sb0, sb1, rb0, rb1, va, vb,
             rs_send, rs_recv, ag_send, ag_recv, sem_a, sem_b, sem_o):
        me = lax.axis_index("i")
        right = lax.rem(me + 1, N_DEV)
        left = lax.rem(me + N_DEV - 1, N_DEV)
        ring_nbr = (right, left)
        ring_col0 = (0, HALF)
        ring_rb = (rb0, rb1)
        ring_sb = (sb0, sb1)

        barrier = pltpu.get_barrier_semaphore()
        for nbr in (left, right):
            pl.semaphore_signal(barrier, inc=1, device_id=(nbr,),
                                device_id_type=pl.DeviceIdType.MESH)
        pl.semaphore_wait(barrier, 2)

        def accum_piece(src_piece, p_row0, p_col0, dest_piece, apply_gelu):
            ca = pltpu.make_async_copy(src_piece, va, sem_a)
            cb = pltpu.make_async_copy(
                p_ref.at[pl.ds(p_row0, PIECE), pl.ds(p_col0, HALF)], vb, sem_b)
            ca.start()
            cb.start()
            ca.wait()
            cb.wait()
            s = va[...] + vb[...]
            va[...] = _gelu(s) if apply_gelu else s
            co = pltpu.make_async_copy(va, dest_piece, sem_o)
            co.start()
            co.wait()

        descs = {}
        ag_descs = {}

        def start_ag(r, t, j):
            if r == 0:
                g = lax.rem(me + 1 - t + N_DEV, N_DEV)
            else:
                g = lax.rem(me - 1 + t + N_DEV, N_DEV)
            piece = out_ref.at[pl.ds(g * CHUNK + j * PIECE, PIECE),
                               pl.ds(ring_col0[r], HALF)]
            d = pltpu.make_async_remote_copy(
                src_ref=piece, dst_ref=piece,
                send_sem=ag_send.at[r, t, j],
                recv_sem=ag_recv.at[r, t, j],
                device_id=(ring_nbr[r],),
                device_id_type=pl.DeviceIdType.MESH,
            )
            d.start()
            ag_descs[(r, t, j)] = d

        def start_send(r, s, j, src_piece):
            d = pltpu.make_async_remote_copy(
                src_ref=src_piece,
                dst_ref=ring_rb[r].at[s, pl.ds(j * PIECE, PIECE), :],
                send_sem=rs_send.at[r, s, j],
                recv_sem=rs_recv.at[r, s, j],
                device_id=(ring_nbr[r],),
                device_id_type=pl.DeviceIdType.MESH,
            )
            d.start()
            descs[(r, s, j)] = d

        for j in range(PIECES):
            for r in (0, 1):
                start_send(r, 0, j, p_ref.at[
                    pl.ds(me * CHUNK + j * PIECE, PIECE),
                    pl.ds(ring_col0[r], HALF)])

        for s in range(N_STEP):
            last = s == N_STEP - 1
            rc = (lax.rem(me - s - 1 + N_DEV, N_DEV),
                  lax.rem(me + s + 1, N_DEV))
            for j in range(PIECES):
                for r in (0, 1):
                    descs[(r, s, j)].wait_recv()
                    src = ring_rb[r].at[s, pl.ds(j * PIECE, PIECE), :]
                    p_row0 = rc[r] * CHUNK + j * PIECE
                    if not last:
                        dest = ring_sb[r].at[s, pl.ds(j * PIECE, PIECE), :]
                        accum_piece(src, p_row0, ring_col0[r], dest, False)
                        start_send(r, s + 1, j, dest)
                    else:
                        dest = out_ref.at[pl.ds(p_row0, PIECE),
                                          pl.ds(ring_col0[r], HALF)]
                        accum_piece(src, p_row0, ring_col0[r], dest, True)
                        start_ag(r, 0, j)
        for d in descs.values():
            d.wait_send()

        for t in range(1, N_STEP):
            for j in range(PIECES):
                for r in (0, 1):
                    ag_descs[(r, t - 1, j)].wait_recv()
                    start_ag(r, t, j)
        for j in range(PIECES):
            for r in (0, 1):
                ag_descs[(r, N_STEP - 1, j)].wait_recv()
        for d in ag_descs.values():
            d.wait_send()

        def _exit(second_barrier):
            for nbr in (left, right):
                pl.semaphore_signal(second_barrier, inc=1, device_id=(nbr,),
                                    device_id_type=pl.DeviceIdType.MESH)
            pl.semaphore_wait(second_barrier, 2)
        pl.run_scoped(_exit, second_barrier=pltpu.SemaphoreType.REGULAR)

    out = pl.pallas_call(
        body,
        out_shape=[
            jax.ShapeDtypeStruct((M, N), jnp.float32),
            jax.ShapeDtypeStruct((N_STEP - 1, CHUNK, HALF), jnp.float32),
            jax.ShapeDtypeStruct((N_STEP - 1, CHUNK, HALF), jnp.float32),
            jax.ShapeDtypeStruct((N_STEP, CHUNK, HALF), jnp.float32),
            jax.ShapeDtypeStruct((N_STEP, CHUNK, HALF), jnp.float32),
        ],
        in_specs=[pl.BlockSpec(memory_space=pl.ANY)],
        out_specs=[pl.BlockSpec(memory_space=pl.ANY)] * 5,
        scratch_shapes=[
            pltpu.MemorySpace.VMEM((PIECE, HALF), jnp.float32),
            pltpu.MemorySpace.VMEM((PIECE, HALF), jnp.float32),
            pltpu.SemaphoreType.DMA((2, N_STEP, PIECES)),
            pltpu.SemaphoreType.DMA((2, N_STEP, PIECES)),
            pltpu.SemaphoreType.DMA((2, N_STEP, PIECES)),
            pltpu.SemaphoreType.DMA((2, N_STEP, PIECES)),
            pltpu.SemaphoreType.DMA,
            pltpu.SemaphoreType.DMA,
            pltpu.SemaphoreType.DMA,
        ],
        compiler_params=pltpu.CompilerParams(collective_id=0),
    )(p)
    return out[0]


# device time: 1194317 ns/iter; 2.0578x vs baseline; 1.0526x over previous
import jax
import jax.numpy as jnp
from jax import lax
from jax.experimental import pallas as pl
from jax.experimental.pallas import tpu as pltpu

N_DEV = 4
M = 4096
N = 8192
HALF = N // 2
CHUNK = M // N_DEV
PIECE = 256
PIECES = CHUNK // PIECE
N_STEP = N_DEV - 1


def _gelu(y):
    c = 0.7978845608028654
    return 0.5 * y * (1.0 + jnp.tanh(c * (y + 0.044715 * y * y * y)))


def kernel(x, w_mat):
    def body(x_ref, w_ref, out_ref, p0, sb0, sb1, rb0, rb1, va, vc,
             rs_send, rs_recv, ag_send, ag_recv, sem_a, sem_c, sem_o):
        me = lax.axis_index("i")
        right = lax.rem(me + 1, N_DEV)
        left = lax.rem(me + N_DEV - 1, N_DEV)
        ring_nbr = (right, left)
        ring_col0 = (0, HALF)
        ring_rb = (rb0, rb1)
        ring_sb = (sb0, sb1)

        def dot_piece(row0, col0, width):
            return jnp.dot(
                x_ref[pl.ds(row0, PIECE), :],
                w_ref[:, pl.ds(col0, width)],
                preferred_element_type=jnp.float32,
            )

        barrier = pltpu.get_barrier_semaphore()
        for nbr in (left, right):
            pl.semaphore_signal(barrier, inc=1, device_id=(nbr,),
                                device_id_type=pl.DeviceIdType.MESH)
        pl.semaphore_wait(barrier, 2)

        def accum_piece(src_piece, row0, col0, dest_piece, apply_gelu):
            ca = pltpu.make_async_copy(src_piece, va, sem_a)
            ca.start()
            pp = dot_piece(row0, col0, HALF)
            ca.wait()
            s = va[...] + pp
            va[...] = _gelu(s) if apply_gelu else s
            co = pltpu.make_async_copy(va, dest_piece, sem_o)
            co.start()
            co.wait()

        descs = {}
        ag_descs = {}

        def start_send(r, s, j, src_piece):
            d = pltpu.make_async_remote_copy(
                src_ref=src_piece,
                dst_ref=ring_rb[r].at[s, pl.ds(j * PIECE, PIECE), :],
                send_sem=rs_send.at[r, s, j],
                recv_sem=rs_recv.at[r, s, j],
                device_id=(ring_nbr[r],),
                device_id_type=pl.DeviceIdType.MESH,
            )
            d.start()
            descs[(r, s, j)] = d

        def start_ag(r, t, j):
            if r == 0:
                g = lax.rem(me + 1 - t + N_DEV, N_DEV)
            else:
                g = lax.rem(me - 1 + t + N_DEV, N_DEV)
            piece = out_ref.at[pl.ds(g * CHUNK + j * PIECE, PIECE),
                               pl.ds(ring_col0[r], HALF)]
            d = pltpu.make_async_remote_copy(
                src_ref=piece, dst_ref=piece,
                send_sem=ag_send.at[r, t, j],
                recv_sem=ag_recv.at[r, t, j],
                device_id=(ring_nbr[r],),
                device_id_type=pl.DeviceIdType.MESH,
            )
            d.start()
            ag_descs[(r, t, j)] = d

        for j in range(PIECES):
            vc[...] = dot_piece(me * CHUNK + j * PIECE, 0, N)
            cp = pltpu.make_async_copy(
                vc, p0.at[pl.ds(j * PIECE, PIECE), :], sem_c)
            cp.start()
            cp.wait()
            for r in (0, 1):
                start_send(r, 0, j, p0.at[pl.ds(j * PIECE, PIECE),
                                          pl.ds(ring_col0[r], HALF)])

        for s in range(N_STEP):
            last = s == N_STEP - 1
            rc = (lax.rem(me - s - 1 + N_DEV, N_DEV),
                  lax.rem(me + s + 1, N_DEV))
            for j in range(PIECES):
                for r in (0, 1):
                    descs[(r, s, j)].wait_recv()
                    src = ring_rb[r].at[s, pl.ds(j * PIECE, PIECE), :]
                    row0 = rc[r] * CHUNK + j * PIECE
                    if not last:
                        dest = ring_sb[r].at[s, pl.ds(j * PIECE, PIECE), :]
                        accum_piece(src, row0, ring_col0[r], dest, False)
                        start_send(r, s + 1, j, dest)
                    else:
                        dest = out_ref.at[pl.ds(row0, PIECE),
                                          pl.ds(ring_col0[r], HALF)]
                        accum_piece(src, row0, ring_col0[r], dest, True)
                        start_ag(r, 0, j)
        for d in descs.values():
            d.wait_send()

        for t in range(1, N_STEP):
            for j in range(PIECES):
                for r in (0, 1):
                    ag_descs[(r, t - 1, j)].wait_recv()
                    start_ag(r, t, j)
        for j in range(PIECES):
            for r in (0, 1):
                ag_descs[(r, N_STEP - 1, j)].wait_recv()
        for d in ag_descs.values():
            d.wait_send()

        def _exit(second_barrier):
            for nbr in (left, right):
                pl.semaphore_signal(second_barrier, inc=1, device_id=(nbr,),
                                    device_id_type=pl.DeviceIdType.MESH)
            pl.semaphore_wait(second_barrier, 2)
        pl.run_scoped(_exit, second_barrier=pltpu.SemaphoreType.REGULAR)

    out = pl.pallas_call(
        body,
        out_shape=[
            jax.ShapeDtypeStruct((M, N), jnp.float32),
            jax.ShapeDtypeStruct((CHUNK, N), jnp.float32),
            jax.ShapeDtypeStruct((N_STEP - 1, CHUNK, HALF), jnp.float32),
            jax.ShapeDtypeStruct((N_STEP - 1, CHUNK, HALF), jnp.float32),
            jax.ShapeDtypeStruct((N_STEP, CHUNK, HALF), jnp.float32),
            jax.ShapeDtypeStruct((N_STEP, CHUNK, HALF), jnp.float32),
        ],
        in_specs=[
            pl.BlockSpec(memory_space=pltpu.MemorySpace.VMEM),
            pl.BlockSpec(memory_space=pltpu.MemorySpace.VMEM),
        ],
        out_specs=[pl.BlockSpec(memory_space=pl.ANY)] * 6,
        scratch_shapes=[
            pltpu.MemorySpace.VMEM((PIECE, HALF), jnp.float32),
            pltpu.MemorySpace.VMEM((PIECE, N), jnp.float32),
            pltpu.SemaphoreType.DMA((2, N_STEP, PIECES)),
            pltpu.SemaphoreType.DMA((2, N_STEP, PIECES)),
            pltpu.SemaphoreType.DMA((2, N_STEP, PIECES)),
            pltpu.SemaphoreType.DMA((2, N_STEP, PIECES)),
            pltpu.SemaphoreType.DMA,
            pltpu.SemaphoreType.DMA,
            pltpu.SemaphoreType.DMA,
        ],
        compiler_params=pltpu.CompilerParams(
            collective_id=0,
            vmem_limit_bytes=100 * 1024 * 1024,
        ),
    )(x, w_mat)
    return out[0]
